# baseline (speedup 1.0000x reference)
import functools

import jax
import jax.numpy as jnp
from jax import lax
from jax.experimental import pallas as pl
from jax.experimental.pallas import tpu as pltpu
from jax.experimental.pallas import tpu_sc as plsc

P, NS, NR, L, K, D = 50000, 5000, 5000, 8, 32, 64
D2 = 2 * D
NOP = NS + NR
NW = 32
OPS_PAD = 10240
OPS_PER_W = OPS_PAD // NW
OPS_PAD_L = 10240
OPS_PER_WL = OPS_PAD_L // NW
PL_TOTAL = P * L
GBLK = 80
GFIRE = 10
GCHUNK = GBLK * GFIRE
NCHUNK = PL_TOTAL // GCHUNK
TP = 1000


def _wid():
    return lax.axis_index("s") * 2 + lax.axis_index("c")


def _sc_gather_rows(table, idx3):
    mesh = plsc.VectorSubcoreMesh(core_axis_name="c", subcore_axis_name="s")

    @functools.partial(
        pl.kernel, mesh=mesh,
        out_type=jax.ShapeDtypeStruct((PL_TOTAL, D2), jnp.float32),
        scratch_types=[
            pltpu.VMEM((GFIRE, GBLK), jnp.int32),
            pltpu.VMEM((GCHUNK, D2), jnp.float32),
            pltpu.SemaphoreType.DMA,
        ],
    )
    def k(table_hbm, idx_hbm, out_hbm, idx_v, rows_v, sem):
        w = _wid()
        nch = (NCHUNK - w + NW - 1) // NW

        def chunk(c, _):
            m = w + c * NW
            pltpu.sync_copy(idx_hbm.at[m], idx_v)
            cps = [
                pltpu.make_async_copy(
                    table_hbm.at[idx_v.at[j]],
                    rows_v.at[pl.ds(j * GBLK, GBLK)], sem)
                for j in range(GFIRE)
            ]
            for cp in cps:
                cp.start()
            for cp in cps:
                cp.wait()
            pltpu.sync_copy(rows_v, out_hbm.at[pl.ds(m * GCHUNK, GCHUNK)])
            return 0

        lax.fori_loop(0, nch, chunk, 0)

    return k(table, idx3)


def _sc_gather_sum(table, idx3):
    mesh = plsc.VectorSubcoreMesh(core_axis_name="c", subcore_axis_name="s")
    NBLK = OPS_PER_W * K // 128
    OPB = 128 // K
    DEPTH = 4

    @functools.partial(
        pl.kernel, mesh=mesh,
        out_type=jax.ShapeDtypeStruct((OPS_PAD, D), jnp.float32),
        scratch_types=[
            pltpu.VMEM((NBLK, 128), jnp.int32),
            pltpu.VMEM((DEPTH * 128, D2), jnp.float32),
            pltpu.VMEM((OPS_PER_W, D), jnp.float32),
        ] + [pltpu.SemaphoreType.DMA] * DEPTH,
    )
    def k(table_hbm, idx_hbm, out_hbm, idx_v, rows_v, acc_v, *sems):
        w = _wid()
        pltpu.sync_copy(idx_hbm.at[w], idx_v)

        def gcopy(b, r):
            return pltpu.make_async_copy(
                table_hbm.at[idx_v.at[b]],
                rows_v.at[pl.ds(r * 128, 128)], sems[r])

        def gsum(b, r):
            for o in range(OPB):
                base = r * 128 + o * K
                orow = b * OPB + o
                for ch in range(D // 16):
                    sl = pl.ds(ch * 16, 16)
                    a0 = rows_v[base + 0, sl]
                    a1 = rows_v[base + 1, sl]
                    for i in range(2, K, 2):
                        a0 = a0 + rows_v[base + i, sl]
                        a1 = a1 + rows_v[base + i + 1, sl]
                    acc_v[orow, sl] = a0 + a1

        for r in range(DEPTH - 1):
            gcopy(r, r).start()

        def outer(g, _):
            for r in range(DEPTH):
                b = g * DEPTH + r
                gcopy(b, r).wait()
                gsum(b, r)

                @pl.when(b + DEPTH - 1 < NBLK)
                def _():
                    gcopy(b + DEPTH - 1, (r + DEPTH - 1) % DEPTH).start()
            return 0

        lax.fori_loop(0, NBLK // DEPTH, outer, 0)
        pltpu.sync_copy(acc_v, out_hbm.at[pl.ds(w * OPS_PER_W, OPS_PER_W)])

    return k(table, idx3)


def _sc_load_sums(traffic_flat, idx2):
    mesh = plsc.VectorSubcoreMesh(core_axis_name="c", subcore_axis_name="s")
    NG = OPS_PER_WL // 16
    GPW = K * 16

    @functools.partial(
        pl.kernel, mesh=mesh,
        out_type=jax.ShapeDtypeStruct((OPS_PAD_L,), jnp.float32),
        scratch_types=[
            pltpu.VMEM((OPS_PER_WL * K,), jnp.int32),
            pltpu.VMEM((GPW,), jnp.float32),
            pltpu.VMEM((OPS_PER_WL,), jnp.float32),
            pltpu.SemaphoreType.DMA,
        ],
    )
    def k(tr_hbm, idx_hbm, out_hbm, idx_v, rows_v, out_v, sem):
        w = _wid()
        pltpu.sync_copy(idx_hbm.at[w], idx_v)

        def per_group(g, _):
            cps = [
                pltpu.make_async_copy(
                    tr_hbm.at[idx_v.at[pl.ds(g * GPW + j * 128, 128)]],
                    rows_v.at[pl.ds(j * 128, 128)], sem)
                for j in range(GPW // 128)
            ]
            for cp in cps:
                cp.start()
            for cp in cps:
                cp.wait()

            def red(kk, a):
                return a + rows_v[pl.ds(kk * 16, 16)]
            out_v[pl.ds(g * 16, 16)] = lax.fori_loop(
                0, K, red, jnp.zeros((16,), jnp.float32))
            return 0

        lax.fori_loop(0, NG, per_group, 0)
        pltpu.sync_copy(out_v, out_hbm.at[pl.ds(w * OPS_PER_WL, OPS_PER_WL)])

    return k(traffic_flat, idx2)


def _full(shape):
    return pl.BlockSpec(shape, lambda i: tuple(0 for _ in shape))


def _tc_path_embed(x, W1, b1, W2, b2):
    def body(x_ref, w1_ref, b1_ref, w2_ref, b2_ref, o_ref):
        h = jnp.maximum(
            jnp.dot(x_ref[...], w1_ref[...],
                    preferred_element_type=jnp.float32) + b1_ref[0], 0.0)
        o_ref[...] = jnp.maximum(
            jnp.dot(h, w2_ref[...],
                    preferred_element_type=jnp.float32) + b2_ref[0], 0.0)

    return pl.pallas_call(
        body,
        grid=(P // TP,),
        in_specs=[
            pl.BlockSpec((TP, 128), lambda i: (i, 0)),
            _full((128, D)), _full((1, D)), _full((D, D)), _full((1, D)),
        ],
        out_specs=pl.BlockSpec((TP, D), lambda i: (i, 0)),
        out_shape=jax.ShapeDtypeStruct((P, D), jnp.float32),
    )(x, W1, b1.reshape(1, D), W2, b2.reshape(1, D))


def _gru_math(x, hx, h, wz, wr, wh, uz, ur, uh, bz, br, bxh, bhh):
    dot = lambda a, b: jnp.dot(a, b, preferred_element_type=jnp.float32)
    z = jax.nn.sigmoid(dot(x, wz) + dot(hx, uz) + bz)
    r = jax.nn.sigmoid(dot(x, wr) + dot(hx, ur) + br)
    n = jnp.tanh(dot(x, wh) + bxh + r * (dot(hx, uh) + bhh))
    return z * h + (1.0 - z) * n


def _tc_gru_scan(og, h0, Ws, Us, bs):
    wz, wr, wh = Ws
    uz, ur, uh = Us

    def body(og_ref, h0_ref, wz_r, wr_r, wh_r, uz_r, ur_r, uh_r, b_r,
             pss_ref, hf_ref):
        zpad = jnp.zeros((TP, D), jnp.float32)
        h = h0_ref[...]
        pss_ref[0] = jnp.concatenate([h, zpad], axis=1)
        for t in range(L):
            x = og_ref[:, t, :]
            h = _gru_math(x, h, h, wz_r[...], wr_r[...], wh_r[...],
                          uz_r[...], ur_r[...], uh_r[...],
                          b_r[0], b_r[1], b_r[2], b_r[3])
            pss_ref[t + 1] = jnp.concatenate([h, zpad], axis=1)
        hf_ref[...] = h

    bstack = jnp.stack(bs)
    return pl.pallas_call(
        body,
        grid=(P // TP,),
        in_specs=[
            pl.BlockSpec((TP, L, D2), lambda i: (i, 0, 0)),
            pl.BlockSpec((TP, D), lambda i: (i, 0)),
            _full((D2, D)), _full((D2, D)), _full((D2, D)),
            _full((D, D)), _full((D, D)), _full((D, D)),
            _full((4, D)),
        ],
        out_specs=[
            pl.BlockSpec((L + 1, TP, D2), lambda i: (0, i, 0)),
            pl.BlockSpec((TP, D), lambda i: (i, 0)),
        ],
        out_shape=[
            jax.ShapeDtypeStruct((L + 1, P, D2), jnp.float32),
            jax.ShapeDtypeStruct((P, D), jnp.float32),
        ],
    )(og, h0, wz, wr, wh, uz, ur, uh, bstack)


def _tc_op_init(caps, sums, W1s, b1s, W2s, b2s):
    TA = 1109111900.0 - 6677.713
    TB = 6677.713

    def body(cap_ref, sum_ref, w1_ref, b1_ref, w2_ref, b2_ref, o_ref):
        cap = cap_ref[...]
        denorm_cap = (cap * (80.0 - 1.0) + 1.0) * 1e9
        load = (TA * sum_ref[...] + K * TB) / denorm_cap
        h = jnp.maximum(cap * w1_ref[0, 0] + load * w1_ref[0, 1]
                        + b1_ref[0, 0], 0.0)
        h = jnp.maximum(
            jnp.dot(h, w2_ref[0], preferred_element_type=jnp.float32)
            + b2_ref[0, 0], 0.0)
        o_ref[...] = jnp.concatenate(
            [h, jnp.zeros((NS, D), jnp.float32)], axis=1)

    return pl.pallas_call(
        body,
        grid=(2,),
        in_specs=[
            pl.BlockSpec((NS, 1), lambda i: (i, 0)),
            pl.BlockSpec((NS, 1), lambda i: (i, 0)),
            pl.BlockSpec((1, 2, D), lambda i: (i, 0, 0)),
            pl.BlockSpec((1, 1, D), lambda i: (i, 0, 0)),
            pl.BlockSpec((1, D, D), lambda i: (i, 0, 0)),
            pl.BlockSpec((1, 1, D), lambda i: (i, 0, 0)),
        ],
        out_specs=pl.BlockSpec((NS, D2), lambda i: (i, 0)),
        out_shape=jax.ShapeDtypeStruct((NOP, D2), jnp.float32),
    )(caps, sums, W1s, b1s, W2s, b2s)


def _tc_op_update(gsum, states, Wstk, Ustk, bstk):
    def body(g_ref, s_ref, w_ref, u_ref, b_ref, o_ref):
        s = s_ref[...]
        hn = _gru_math(
            g_ref[...], s, s[:, :D],
            w_ref[0, 0], w_ref[0, 1], w_ref[0, 2],
            u_ref[0, 0], u_ref[0, 1], u_ref[0, 2],
            b_ref[0, 0], b_ref[0, 1], b_ref[0, 2], b_ref[0, 3])
        o_ref[...] = jnp.concatenate(
            [hn, jnp.zeros((NS, D), jnp.float32)], axis=1)

    return pl.pallas_call(
        body,
        grid=(2,),
        in_specs=[
            pl.BlockSpec((NS, D), lambda i: (i, 0)),
            pl.BlockSpec((NS, D2), lambda i: (i, 0)),
            pl.BlockSpec((1, 3, D, D), lambda i: (i, 0, 0, 0)),
            pl.BlockSpec((1, 3, D2, D), lambda i: (i, 0, 0, 0)),
            pl.BlockSpec((1, 4, D), lambda i: (i, 0, 0)),
        ],
        out_specs=pl.BlockSpec((NS, D2), lambda i: (i, 0)),
        out_shape=jax.ShapeDtypeStruct((NOP, D2), jnp.float32),
    )(gsum, states, Wstk, Ustk, bstk)


def _tc_readout(pss, W1, b1, W2, b2, W3, b3):
    def body(pss_ref, w1_ref, b1_ref, w2_ref, b2_ref, w3_ref, b3_ref, o_ref):
        dot = lambda a, b: jnp.dot(a, b, preferred_element_type=jnp.float32)
        acc = jnp.zeros((TP, 1), jnp.float32)
        for t in range(L):
            x = pss_ref[t + 1]
            h = jnp.maximum(dot(x, w1_ref[...]) + b1_ref[0], 0.0)
            h = jnp.maximum(dot(h, w2_ref[...]) + b2_ref[0], 0.0)
            acc = acc + jax.nn.softplus(dot(h, w3_ref[...]) + b3_ref[0])
        o_ref[...] = jnp.log(acc)

    W1 = _padrows(W1)
    return pl.pallas_call(
        body,
        grid=(P // TP,),
        in_specs=[
            pl.BlockSpec((L + 1, TP, D2), lambda i: (0, i, 0)),
            _full((D2, 32)), _full((1, 32)),
            _full((32, 16)), _full((1, 16)),
            _full((16, 1)), _full((1, 1)),
        ],
        out_specs=pl.BlockSpec((TP, 1), lambda i: (i, 0)),
        out_shape=jax.ShapeDtypeStruct((P, 1), jnp.float32),
    )(pss, W1, b1.reshape(1, 32), W2, b2.reshape(1, 16),
      W3, b3.reshape(1, 1))


def _padrows(W):
    return jnp.concatenate([W, jnp.zeros((D2 - D,) + W.shape[1:], W.dtype)],
                           axis=0)


def _split_gru(Wg, Ug, bg, pad_x):
    Ws = (Wg[:, :D], Wg[:, D:2 * D], Wg[:, 2 * D:])
    if pad_x:
        Ws = tuple(_padrows(w) for w in Ws)
    Us = (Ug[:, :D], Ug[:, D:2 * D], Ug[:, 2 * D:])
    bz = bg[0, :D] + bg[1, :D]
    br = bg[0, D:2 * D] + bg[1, D:2 * D]
    bxh = bg[0, 2 * D:]
    bhh = bg[1, 2 * D:]
    return Ws, Us, (bz, br, bxh, bhh)


def _stack_gru(sW, sU, sb, rW, rU, rb):
    sWs, sUs, sbs = _split_gru(sW, sU, sb, False)
    rWs, rUs, rbs = _split_gru(rW, rU, rb, False)
    sUs = tuple(_padrows(u) for u in sUs)
    rUs = tuple(_padrows(u) for u in rUs)
    Wstk = jnp.stack([jnp.stack(sWs), jnp.stack(rWs)])
    Ustk = jnp.stack([jnp.stack(sUs), jnp.stack(rUs)])
    bstk = jnp.stack([jnp.stack(sbs), jnp.stack(rbs)])
    return Wstk, Ustk, bstk


def kernel(traffic, packets, packet_size, ipg, s_capacity, r_capacity,
           path_to_s_op, path_to_r_op, ops_to_path,
           pe_W1, pe_b1, pe_W2, pe_b2, se_W1, se_b1, se_W2, se_b2,
           re_W1, re_b1, re_W2, re_b2, pg_W, pg_U, pg_b,
           sg_W, sg_U, sg_b, rg_W, rg_U, rg_b,
           ro_W1, ro_b1, ro_W2, ro_b2, ro_W3, ro_b3):
    og_idx = ops_to_path.reshape(NCHUNK, GFIRE, GBLK).astype(jnp.int32)

    sp = path_to_s_op[:, :, 0]
    rp = path_to_r_op[:, :, 0]
    pad_l = jnp.zeros((OPS_PAD_L - NOP, K), jnp.int32)
    tr_idx = (jnp.concatenate([sp, rp, pad_l], axis=0)
              .reshape(NW, OPS_PER_WL // 16, 16, K)
              .transpose(0, 1, 3, 2)
              .reshape(NW, OPS_PER_WL * K))

    pad = jnp.zeros((OPS_PAD - NOP, K), jnp.int32)
    sflat = path_to_s_op[:, :, 1] * P + sp
    rflat = path_to_r_op[:, :, 1] * P + rp
    pss_idx = jnp.concatenate([sflat, rflat, pad], axis=0).reshape(
        NW, OPS_PER_W * K // 128, 128)

    x = jnp.concatenate([traffic, packets, packet_size, ipg,
                         jnp.zeros((P, 15), jnp.float32)], axis=1)
    pe_W1p = jnp.concatenate([pe_W1, jnp.zeros((15, D), jnp.float32)], axis=0)
    path_state = _tc_path_embed(x, pe_W1p, pe_b1, pe_W2, pe_b2)

    raw_sums = _sc_load_sums(traffic.reshape(P), tr_idx)

    caps = jnp.concatenate([s_capacity, r_capacity], axis=0)
    W1s = jnp.stack([se_W1, re_W1])
    b1s = jnp.stack([se_b1, re_b1]).reshape(2, 1, D)
    W2s = jnp.stack([se_W2, re_W2])
    b2s = jnp.stack([se_b2, re_b2]).reshape(2, 1, D)
    states = _tc_op_init(caps, raw_sums.reshape(OPS_PAD_L, 1)[:NOP],
                         W1s, b1s, W2s, b2s)

    pWs, pUs, pbs = _split_gru(pg_W, pg_U, pg_b, True)
    Wstk, Ustk, bstk = _stack_gru(sg_W, sg_U, sg_b, rg_W, rg_U, rg_b)

    pss = None
    hcur = path_state
    for it in range(8):
        og = _sc_gather_rows(states, og_idx).reshape(P, L, D2)
        pss, hcur = _tc_gru_scan(og, hcur, pWs, pUs, pbs)
        gsum = _sc_gather_sum(pss.reshape((L + 1) * P, D2), pss_idx)
        states = _tc_op_update(gsum, states, Wstk, Ustk, bstk)

    return _tc_readout(pss, ro_W1, ro_b1, ro_W2, ro_b2, ro_W3, ro_b3)


# speedup vs baseline: 1.0333x; 1.0333x over previous
import functools

import jax
import jax.numpy as jnp
from jax import lax
from jax.experimental import pallas as pl
from jax.experimental.pallas import tpu as pltpu
from jax.experimental.pallas import tpu_sc as plsc

P, NS, NR, L, K, D = 50000, 5000, 5000, 8, 32, 64
D2 = 2 * D
NOP = NS + NR
NW = 32
OPS_PAD = 10240
OPS_PER_W = OPS_PAD // NW
OPS_PAD_L = 10240
OPS_PER_WL = OPS_PAD_L // NW
PL_TOTAL = P * L
GBLK = 80
GFIRE = 10
GCHUNK = GBLK * GFIRE
NCHUNK = PL_TOTAL // GCHUNK
TP = 1000


def _wid():
    return lax.axis_index("s") * 2 + lax.axis_index("c")


def _sc_gather_rows(table, idx3):
    mesh = plsc.VectorSubcoreMesh(core_axis_name="c", subcore_axis_name="s")

    @functools.partial(
        pl.kernel, mesh=mesh,
        out_type=jax.ShapeDtypeStruct((PL_TOTAL, D2), jnp.float32),
        scratch_types=[
            pltpu.VMEM((GFIRE, GBLK), jnp.int32),
            pltpu.VMEM((GCHUNK, D2), jnp.float32),
            pltpu.SemaphoreType.DMA,
        ],
    )
    def k(table_hbm, idx_hbm, out_hbm, idx_v, rows_v, sem):
        w = _wid()
        nch = (NCHUNK - w + NW - 1) // NW

        def chunk(c, _):
            m = w + c * NW
            pltpu.sync_copy(idx_hbm.at[m], idx_v)
            cps = [
                pltpu.make_async_copy(
                    table_hbm.at[idx_v.at[j]],
                    rows_v.at[pl.ds(j * GBLK, GBLK)], sem)
                for j in range(GFIRE)
            ]
            for cp in cps:
                cp.start()
            for cp in cps:
                cp.wait()
            pltpu.sync_copy(rows_v, out_hbm.at[pl.ds(m * GCHUNK, GCHUNK)])
            return 0

        lax.fori_loop(0, nch, chunk, 0)

    return k(table, idx3)


def _sc_gather_sum(table, idx3):
    mesh = plsc.VectorSubcoreMesh(core_axis_name="c", subcore_axis_name="s")
    BLK = 64
    NBLK = OPS_PER_W * K // BLK
    OPB = BLK // K
    DEPTH = 8

    @functools.partial(
        pl.kernel, mesh=mesh,
        out_type=jax.ShapeDtypeStruct((OPS_PAD, D), jnp.float32),
        scratch_types=[
            pltpu.VMEM((NBLK, BLK), jnp.int32),
            pltpu.VMEM((DEPTH * BLK, D2), jnp.float32),
            pltpu.VMEM((OPS_PER_W, D), jnp.float32),
        ] + [pltpu.SemaphoreType.DMA] * DEPTH,
    )
    def k(table_hbm, idx_hbm, out_hbm, idx_v, rows_v, acc_v, *sems):
        w = _wid()
        pltpu.sync_copy(idx_hbm.at[w], idx_v)

        def gcopy(b, r):
            return pltpu.make_async_copy(
                table_hbm.at[idx_v.at[b]],
                rows_v.at[pl.ds(r * BLK, BLK)], sems[r])

        def gsum(b, r):
            for o in range(OPB):
                base = r * BLK + o * K
                orow = b * OPB + o
                for ch in range(D // 16):
                    sl = pl.ds(ch * 16, 16)
                    a0 = rows_v[base + 0, sl]
                    a1 = rows_v[base + 1, sl]
                    for i in range(2, K, 2):
                        a0 = a0 + rows_v[base + i, sl]
                        a1 = a1 + rows_v[base + i + 1, sl]
                    acc_v[orow, sl] = a0 + a1

        for r in range(DEPTH - 1):
            gcopy(r, r).start()

        def outer(g, _):
            for r in range(DEPTH):
                b = g * DEPTH + r
                gcopy(b, r).wait()
                gsum(b, r)

                @pl.when(b + DEPTH - 1 < NBLK)
                def _():
                    gcopy(b + DEPTH - 1, (r + DEPTH - 1) % DEPTH).start()
            return 0

        lax.fori_loop(0, NBLK // DEPTH, outer, 0)
        pltpu.sync_copy(acc_v, out_hbm.at[pl.ds(w * OPS_PER_W, OPS_PER_W)])

    return k(table, idx3)


def _sc_load_sums(traffic_flat, idx2):
    mesh = plsc.VectorSubcoreMesh(core_axis_name="c", subcore_axis_name="s")
    NG = OPS_PER_WL // 16
    GPW = K * 16

    @functools.partial(
        pl.kernel, mesh=mesh,
        out_type=jax.ShapeDtypeStruct((OPS_PAD_L,), jnp.float32),
        scratch_types=[
            pltpu.VMEM((OPS_PER_WL * K,), jnp.int32),
            pltpu.VMEM((GPW,), jnp.float32),
            pltpu.VMEM((OPS_PER_WL,), jnp.float32),
            pltpu.SemaphoreType.DMA,
        ],
    )
    def k(tr_hbm, idx_hbm, out_hbm, idx_v, rows_v, out_v, sem):
        w = _wid()
        pltpu.sync_copy(idx_hbm.at[w], idx_v)

        def per_group(g, _):
            cps = [
                pltpu.make_async_copy(
                    tr_hbm.at[idx_v.at[pl.ds(g * GPW + j * 128, 128)]],
                    rows_v.at[pl.ds(j * 128, 128)], sem)
                for j in range(GPW // 128)
            ]
            for cp in cps:
                cp.start()
            for cp in cps:
                cp.wait()

            def red(kk, a):
                return a + rows_v[pl.ds(kk * 16, 16)]
            out_v[pl.ds(g * 16, 16)] = lax.fori_loop(
                0, K, red, jnp.zeros((16,), jnp.float32))
            return 0

        lax.fori_loop(0, NG, per_group, 0)
        pltpu.sync_copy(out_v, out_hbm.at[pl.ds(w * OPS_PER_WL, OPS_PER_WL)])

    return k(traffic_flat, idx2)


def _full(shape):
    return pl.BlockSpec(shape, lambda i: tuple(0 for _ in shape))


def _tc_path_embed(x, W1, b1, W2, b2):
    def body(x_ref, w1_ref, b1_ref, w2_ref, b2_ref, o_ref):
        h = jnp.maximum(
            jnp.dot(x_ref[...], w1_ref[...],
                    preferred_element_type=jnp.float32) + b1_ref[0], 0.0)
        o_ref[...] = jnp.maximum(
            jnp.dot(h, w2_ref[...],
                    preferred_element_type=jnp.float32) + b2_ref[0], 0.0)

    return pl.pallas_call(
        body,
        grid=(P // TP,),
        in_specs=[
            pl.BlockSpec((TP, 128), lambda i: (i, 0)),
            _full((128, D)), _full((1, D)), _full((D, D)), _full((1, D)),
        ],
        out_specs=pl.BlockSpec((TP, D), lambda i: (i, 0)),
        out_shape=jax.ShapeDtypeStruct((P, D), jnp.float32),
    )(x, W1, b1.reshape(1, D), W2, b2.reshape(1, D))


def _gru_math(x, hx, h, wz, wr, wh, uz, ur, uh, bz, br, bxh, bhh):
    dot = lambda a, b: jnp.dot(a, b, preferred_element_type=jnp.float32)
    z = jax.nn.sigmoid(dot(x, wz) + dot(hx, uz) + bz)
    r = jax.nn.sigmoid(dot(x, wr) + dot(hx, ur) + br)
    n = jnp.tanh(dot(x, wh) + bxh + r * (dot(hx, uh) + bhh))
    return z * h + (1.0 - z) * n


def _tc_gru_scan(og, h0, Ws, Us, bs):
    wz, wr, wh = Ws
    uz, ur, uh = Us

    def body(og_ref, h0_ref, wz_r, wr_r, wh_r, uz_r, ur_r, uh_r, b_r,
             pss_ref, hf_ref):
        zpad = jnp.zeros((TP, D), jnp.float32)
        h = h0_ref[...]
        pss_ref[0] = jnp.concatenate([h, zpad], axis=1)
        for t in range(L):
            x = og_ref[:, t, :]
            h = _gru_math(x, h, h, wz_r[...], wr_r[...], wh_r[...],
                          uz_r[...], ur_r[...], uh_r[...],
                          b_r[0], b_r[1], b_r[2], b_r[3])
            pss_ref[t + 1] = jnp.concatenate([h, zpad], axis=1)
        hf_ref[...] = h

    bstack = jnp.stack(bs)
    return pl.pallas_call(
        body,
        grid=(P // TP,),
        in_specs=[
            pl.BlockSpec((TP, L, D2), lambda i: (i, 0, 0)),
            pl.BlockSpec((TP, D), lambda i: (i, 0)),
            _full((D2, D)), _full((D2, D)), _full((D2, D)),
            _full((D, D)), _full((D, D)), _full((D, D)),
            _full((4, D)),
        ],
        out_specs=[
            pl.BlockSpec((L + 1, TP, D2), lambda i: (0, i, 0)),
            pl.BlockSpec((TP, D), lambda i: (i, 0)),
        ],
        out_shape=[
            jax.ShapeDtypeStruct((L + 1, P, D2), jnp.float32),
            jax.ShapeDtypeStruct((P, D), jnp.float32),
        ],
    )(og, h0, wz, wr, wh, uz, ur, uh, bstack)


def _tc_op_init(caps, sums, W1s, b1s, W2s, b2s):
    TA = 1109111900.0 - 6677.713
    TB = 6677.713

    def body(cap_ref, sum_ref, w1_ref, b1_ref, w2_ref, b2_ref, o_ref):
        cap = cap_ref[...]
        denorm_cap = (cap * (80.0 - 1.0) + 1.0) * 1e9
        load = (TA * sum_ref[...] + K * TB) / denorm_cap
        h = jnp.maximum(cap * w1_ref[0, 0] + load * w1_ref[0, 1]
                        + b1_ref[0, 0], 0.0)
        h = jnp.maximum(
            jnp.dot(h, w2_ref[0], preferred_element_type=jnp.float32)
            + b2_ref[0, 0], 0.0)
        o_ref[...] = jnp.concatenate(
            [h, jnp.zeros((NS, D), jnp.float32)], axis=1)

    return pl.pallas_call(
        body,
        grid=(2,),
        in_specs=[
            pl.BlockSpec((NS, 1), lambda i: (i, 0)),
            pl.BlockSpec((NS, 1), lambda i: (i, 0)),
            pl.BlockSpec((1, 2, D), lambda i: (i, 0, 0)),
            pl.BlockSpec((1, 1, D), lambda i: (i, 0, 0)),
            pl.BlockSpec((1, D, D), lambda i: (i, 0, 0)),
            pl.BlockSpec((1, 1, D), lambda i: (i, 0, 0)),
        ],
        out_specs=pl.BlockSpec((NS, D2), lambda i: (i, 0)),
        out_shape=jax.ShapeDtypeStruct((NOP, D2), jnp.float32),
    )(caps, sums, W1s, b1s, W2s, b2s)


def _tc_op_update(gsum, states, Wstk, Ustk, bstk):
    def body(g_ref, s_ref, w_ref, u_ref, b_ref, o_ref):
        s = s_ref[...]
        hn = _gru_math(
            g_ref[...], s, s[:, :D],
            w_ref[0, 0], w_ref[0, 1], w_ref[0, 2],
            u_ref[0, 0], u_ref[0, 1], u_ref[0, 2],
            b_ref[0, 0], b_ref[0, 1], b_ref[0, 2], b_ref[0, 3])
        o_ref[...] = jnp.concatenate(
            [hn, jnp.zeros((NS, D), jnp.float32)], axis=1)

    return pl.pallas_call(
        body,
        grid=(2,),
        in_specs=[
            pl.BlockSpec((NS, D), lambda i: (i, 0)),
            pl.BlockSpec((NS, D2), lambda i: (i, 0)),
            pl.BlockSpec((1, 3, D, D), lambda i: (i, 0, 0, 0)),
            pl.BlockSpec((1, 3, D2, D), lambda i: (i, 0, 0, 0)),
            pl.BlockSpec((1, 4, D), lambda i: (i, 0, 0)),
        ],
        out_specs=pl.BlockSpec((NS, D2), lambda i: (i, 0)),
        out_shape=jax.ShapeDtypeStruct((NOP, D2), jnp.float32),
    )(gsum, states, Wstk, Ustk, bstk)


def _tc_readout(pss, W1, b1, W2, b2, W3, b3):
    def body(pss_ref, w1_ref, b1_ref, w2_ref, b2_ref, w3_ref, b3_ref, o_ref):
        dot = lambda a, b: jnp.dot(a, b, preferred_element_type=jnp.float32)
        acc = jnp.zeros((TP, 1), jnp.float32)
        for t in range(L):
            x = pss_ref[t + 1]
            h = jnp.maximum(dot(x, w1_ref[...]) + b1_ref[0], 0.0)
            h = jnp.maximum(dot(h, w2_ref[...]) + b2_ref[0], 0.0)
            acc = acc + jax.nn.softplus(dot(h, w3_ref[...]) + b3_ref[0])
        o_ref[...] = jnp.log(acc)

    W1 = _padrows(W1)
    return pl.pallas_call(
        body,
        grid=(P // TP,),
        in_specs=[
            pl.BlockSpec((L + 1, TP, D2), lambda i: (0, i, 0)),
            _full((D2, 32)), _full((1, 32)),
            _full((32, 16)), _full((1, 16)),
            _full((16, 1)), _full((1, 1)),
        ],
        out_specs=pl.BlockSpec((TP, 1), lambda i: (i, 0)),
        out_shape=jax.ShapeDtypeStruct((P, 1), jnp.float32),
    )(pss, W1, b1.reshape(1, 32), W2, b2.reshape(1, 16),
      W3, b3.reshape(1, 1))


def _padrows(W):
    return jnp.concatenate([W, jnp.zeros((D2 - D,) + W.shape[1:], W.dtype)],
                           axis=0)


def _split_gru(Wg, Ug, bg, pad_x):
    Ws = (Wg[:, :D], Wg[:, D:2 * D], Wg[:, 2 * D:])
    if pad_x:
        Ws = tuple(_padrows(w) for w in Ws)
    Us = (Ug[:, :D], Ug[:, D:2 * D], Ug[:, 2 * D:])
    bz = bg[0, :D] + bg[1, :D]
    br = bg[0, D:2 * D] + bg[1, D:2 * D]
    bxh = bg[0, 2 * D:]
    bhh = bg[1, 2 * D:]
    return Ws, Us, (bz, br, bxh, bhh)


def _stack_gru(sW, sU, sb, rW, rU, rb):
    sWs, sUs, sbs = _split_gru(sW, sU, sb, False)
    rWs, rUs, rbs = _split_gru(rW, rU, rb, False)
    sUs = tuple(_padrows(u) for u in sUs)
    rUs = tuple(_padrows(u) for u in rUs)
    Wstk = jnp.stack([jnp.stack(sWs), jnp.stack(rWs)])
    Ustk = jnp.stack([jnp.stack(sUs), jnp.stack(rUs)])
    bstk = jnp.stack([jnp.stack(sbs), jnp.stack(rbs)])
    return Wstk, Ustk, bstk


def kernel(traffic, packets, packet_size, ipg, s_capacity, r_capacity,
           path_to_s_op, path_to_r_op, ops_to_path,
           pe_W1, pe_b1, pe_W2, pe_b2, se_W1, se_b1, se_W2, se_b2,
           re_W1, re_b1, re_W2, re_b2, pg_W, pg_U, pg_b,
           sg_W, sg_U, sg_b, rg_W, rg_U, rg_b,
           ro_W1, ro_b1, ro_W2, ro_b2, ro_W3, ro_b3):
    og_idx = ops_to_path.reshape(NCHUNK, GFIRE, GBLK).astype(jnp.int32)

    sp = path_to_s_op[:, :, 0]
    rp = path_to_r_op[:, :, 0]
    pad_l = jnp.zeros((OPS_PAD_L - NOP, K), jnp.int32)
    tr_idx = (jnp.concatenate([sp, rp, pad_l], axis=0)
              .reshape(NW, OPS_PER_WL // 16, 16, K)
              .transpose(0, 1, 3, 2)
              .reshape(NW, OPS_PER_WL * K))

    pad = jnp.zeros((OPS_PAD - NOP, K), jnp.int32)
    sflat = path_to_s_op[:, :, 1] * P + sp
    rflat = path_to_r_op[:, :, 1] * P + rp
    pss_idx = jnp.concatenate([sflat, rflat, pad], axis=0).reshape(
        NW, OPS_PER_W * K // 64, 64)

    x = jnp.concatenate([traffic, packets, packet_size, ipg,
                         jnp.zeros((P, 15), jnp.float32)], axis=1)
    pe_W1p = jnp.concatenate([pe_W1, jnp.zeros((15, D), jnp.float32)], axis=0)
    path_state = _tc_path_embed(x, pe_W1p, pe_b1, pe_W2, pe_b2)

    raw_sums = _sc_load_sums(traffic.reshape(P), tr_idx)

    caps = jnp.concatenate([s_capacity, r_capacity], axis=0)
    W1s = jnp.stack([se_W1, re_W1])
    b1s = jnp.stack([se_b1, re_b1]).reshape(2, 1, D)
    W2s = jnp.stack([se_W2, re_W2])
    b2s = jnp.stack([se_b2, re_b2]).reshape(2, 1, D)
    states = _tc_op_init(caps, raw_sums.reshape(OPS_PAD_L, 1)[:NOP],
                         W1s, b1s, W2s, b2s)

    pWs, pUs, pbs = _split_gru(pg_W, pg_U, pg_b, True)
    Wstk, Ustk, bstk = _stack_gru(sg_W, sg_U, sg_b, rg_W, rg_U, rg_b)

    pss = None
    hcur = path_state
    for it in range(8):
        og = _sc_gather_rows(states, og_idx).reshape(P, L, D2)
        pss, hcur = _tc_gru_scan(og, hcur, pWs, pUs, pbs)
        gsum = _sc_gather_sum(pss.reshape((L + 1) * P, D2), pss_idx)
        states = _tc_op_update(gsum, states, Wstk, Ustk, bstk)

    return _tc_readout(pss, ro_W1, ro_b1, ro_W2, ro_b2, ro_W3, ro_b3)
